# baseline (speedup 1.0000x reference)
import functools

import jax
import jax.numpy as jnp
from jax import lax
from jax.experimental import pallas as pl
from jax.experimental.pallas import tpu as pltpu
from jax.experimental.pallas import tpu_sc as plsc

_MARGIN = 0.2
_NC = 2
_NS = 16
_NW = _NC * _NS
_L = 16
_CHUNK = 32
_NBUF = 8


def _triplet_loss_body(t_per_w, n_chunks, d,
                       emb_a, emb_p, emb_n, aidx, pidx, nidx, out,
                       *refs):
    bufs = refs[3:3 + 3 * _NBUF]
    aidx_v, pidx_v, nidx_v = refs[0:3]
    vacc_v = refs[3 + 3 * _NBUF]
    sem_i = refs[4 + 3 * _NBUF]
    sems = refs[5 + 3 * _NBUF:]
    bufsets = tuple((bufs[3 * i], bufs[3 * i + 1], bufs[3 * i + 2], sems[i])
                    for i in range(_NBUF))

    wid = lax.axis_index("s") * _NC + lax.axis_index("c")
    base = wid * t_per_w

    ci0 = pltpu.async_copy(aidx.at[pl.ds(base, t_per_w)], aidx_v, sem_i)
    ci1 = pltpu.async_copy(pidx.at[pl.ds(base, t_per_w)], pidx_v, sem_i)
    ci2 = pltpu.async_copy(nidx.at[pl.ds(base, t_per_w)], nidx_v, sem_i)
    ci0.wait()
    ci1.wait()
    ci2.wait()

    def copies(k, bs):
        ab, pb, nb, sem = bs
        off = k * _CHUNK
        return (
            pltpu.make_async_copy(emb_a.at[aidx_v.at[pl.ds(off, _CHUNK)]], ab, sem),
            pltpu.make_async_copy(emb_p.at[pidx_v.at[pl.ds(off, _CHUNK)]], pb, sem),
            pltpu.make_async_copy(emb_n.at[nidx_v.at[pl.ds(off, _CHUNK)]], nb, sem),
        )

    def issue(k, bs):
        for c in copies(k, bs):
            c.start()

    def drain(k, bs):
        for c in copies(k, bs):
            c.wait()

    lane = lax.iota(jnp.int32, _L)
    is_last = lane == _L - 1

    def compute(bs, vacc):
        ab, pb, nb, _ = bs

        def t_body(t, vacc):
            acc = jnp.zeros((_L,), jnp.float32)
            for jg in range(d // _L):
                sl = pl.ds(jg * _L, _L)
                a = ab[t, sl]
                p = pb[t, sl]
                n = nb[t, sl]
                dp = a - p
                dn = a - n
                acc = acc + (dp * dp - dn * dn)
            s = plsc.cumsum(acc)
            contrib = jnp.maximum(s + _MARGIN, 0.0)
            return vacc + jnp.where(is_last, contrib, 0.0)

        return plsc.parallel_loop(0, _CHUNK, unroll=4, carry=vacc)(t_body)

    for b in range(_NBUF - 1):
        issue(b, bufsets[b])

    def round_body(j, vacc):
        k0 = _NBUF * j
        for b in range(_NBUF):
            k = k0 + b
            kn = k + _NBUF - 1
            bn = (b + _NBUF - 1) % _NBUF
            @pl.when(kn < n_chunks)
            def _():
                issue(kn, bufsets[bn])
            drain(k, bufsets[b])
            vacc = compute(bufsets[b], vacc)
        return vacc

    vacc = lax.fori_loop(0, n_chunks // _NBUF, round_body,
                         jnp.zeros((_L,), jnp.float32))
    vacc_v[...] = vacc
    pltpu.sync_copy(vacc_v, out.at[wid])


def kernel(embeddings, target, triplets):
    del target
    t = triplets.shape[0]
    d = embeddings.shape[2]
    t_per_w = t // _NW
    n_chunks = t_per_w // _CHUNK

    mesh = plsc.VectorSubcoreMesh(core_axis_name="c", subcore_axis_name="s",
                                  num_cores=_NC, num_subcores=_NS)
    body = functools.partial(_triplet_loss_body, t_per_w, n_chunks, d)
    run = pl.kernel(
        body,
        out_type=jax.ShapeDtypeStruct((_NW, _L), jnp.float32),
        mesh=mesh,
        compiler_params=pltpu.CompilerParams(needs_layout_passes=False),
        scratch_types=(
            [pltpu.VMEM((t_per_w,), jnp.int32)] * 3
            + [pltpu.VMEM((_CHUNK, d), jnp.float32)] * (3 * _NBUF)
            + [pltpu.VMEM((_L,), jnp.float32)]
            + [pltpu.SemaphoreType.DMA] * (1 + _NBUF)
        ),
    )
    partials = run(embeddings[0], embeddings[1], embeddings[2],
                   triplets[:, 0], triplets[:, 1], triplets[:, 2])
    loss = jnp.sum(partials) / jnp.float32(t)
    return (loss, t)


# speedup vs baseline: 1.2631x; 1.2631x over previous
import functools

import jax
import jax.numpy as jnp
from jax import lax
from jax.experimental import pallas as pl
from jax.experimental.pallas import tpu as pltpu
from jax.experimental.pallas import tpu_sc as plsc

_MARGIN = 0.2
_NC = 2
_NS = 16
_NW = _NC * _NS
_L = 16
_CHUNK = 64
_NBUF = 4


def _triplet_loss_body(t_per_w, n_chunks, d,
                       emb_a, emb_p, emb_n, aidx, pidx, nidx, out,
                       *refs):
    bufs = refs[3:3 + 3 * _NBUF]
    aidx_v, pidx_v, nidx_v = refs[0:3]
    vacc_v = refs[3 + 3 * _NBUF]
    sem_i = refs[4 + 3 * _NBUF]
    sems = refs[5 + 3 * _NBUF:]
    bufsets = tuple((bufs[3 * i], bufs[3 * i + 1], bufs[3 * i + 2], sems[i])
                    for i in range(_NBUF))

    wid = lax.axis_index("s") * _NC + lax.axis_index("c")
    base = wid * t_per_w

    ci0 = pltpu.async_copy(aidx.at[pl.ds(base, t_per_w)], aidx_v, sem_i)
    ci1 = pltpu.async_copy(pidx.at[pl.ds(base, t_per_w)], pidx_v, sem_i)
    ci2 = pltpu.async_copy(nidx.at[pl.ds(base, t_per_w)], nidx_v, sem_i)
    ci0.wait()
    ci1.wait()
    ci2.wait()

    def copies(k, bs):
        ab, pb, nb, sem = bs
        off = k * _CHUNK
        return (
            pltpu.make_async_copy(emb_a.at[aidx_v.at[pl.ds(off, _CHUNK)]], ab, sem),
            pltpu.make_async_copy(emb_p.at[pidx_v.at[pl.ds(off, _CHUNK)]], pb, sem),
            pltpu.make_async_copy(emb_n.at[nidx_v.at[pl.ds(off, _CHUNK)]], nb, sem),
        )

    def issue(k, bs):
        for c in copies(k, bs):
            c.start()

    def drain(k, bs):
        for c in copies(k, bs):
            c.wait()

    lane = lax.iota(jnp.int32, _L)
    is_last = lane == _L - 1

    def compute(bs, vacc):
        ab, pb, nb, _ = bs

        def t_body(t, vacc):
            acc = jnp.zeros((_L,), jnp.float32)
            for jg in range(d // _L):
                sl = pl.ds(jg * _L, _L)
                a = ab[t, sl]
                p = pb[t, sl]
                n = nb[t, sl]
                dp = a - p
                dn = a - n
                acc = acc + (dp * dp - dn * dn)
            s = plsc.cumsum(acc)
            contrib = jnp.maximum(s + _MARGIN, 0.0)
            return vacc + jnp.where(is_last, contrib, 0.0)

        return plsc.parallel_loop(0, _CHUNK, unroll=4, carry=vacc)(t_body)

    for b in range(_NBUF - 1):
        issue(b, bufsets[b])

    def round_body(j, vacc):
        k0 = _NBUF * j
        for b in range(_NBUF):
            k = k0 + b
            kn = k + _NBUF - 1
            bn = (b + _NBUF - 1) % _NBUF
            @pl.when(kn < n_chunks)
            def _():
                issue(kn, bufsets[bn])
            drain(k, bufsets[b])
            vacc = compute(bufsets[b], vacc)
        return vacc

    vacc = lax.fori_loop(0, n_chunks // _NBUF, round_body,
                         jnp.zeros((_L,), jnp.float32))
    vacc_v[...] = vacc
    pltpu.sync_copy(vacc_v, out.at[wid])


def kernel(embeddings, target, triplets):
    del target
    t = triplets.shape[0]
    d = embeddings.shape[2]
    t_per_w = t // _NW
    n_chunks = t_per_w // _CHUNK

    mesh = plsc.VectorSubcoreMesh(core_axis_name="c", subcore_axis_name="s",
                                  num_cores=_NC, num_subcores=_NS)
    body = functools.partial(_triplet_loss_body, t_per_w, n_chunks, d)
    run = pl.kernel(
        body,
        out_type=jax.ShapeDtypeStruct((_NW, _L), jnp.float32),
        mesh=mesh,
        compiler_params=pltpu.CompilerParams(needs_layout_passes=False),
        scratch_types=(
            [pltpu.VMEM((t_per_w,), jnp.int32)] * 3
            + [pltpu.VMEM((_CHUNK, d), jnp.float32)] * (3 * _NBUF)
            + [pltpu.VMEM((_L,), jnp.float32)]
            + [pltpu.SemaphoreType.DMA] * (1 + _NBUF)
        ),
    )
    partials = run(embeddings[0], embeddings[1], embeddings[2],
                   triplets[:, 0], triplets[:, 1], triplets[:, 2])
    loss = jnp.sum(partials) / jnp.float32(t)
    return (loss, t)


# speedup vs baseline: 1.2702x; 1.0056x over previous
import functools

import jax
import jax.numpy as jnp
from jax import lax
from jax.experimental import pallas as pl
from jax.experimental.pallas import tpu as pltpu
from jax.experimental.pallas import tpu_sc as plsc

_MARGIN = 0.2
_NC = 2
_NS = 16
_NW = _NC * _NS
_L = 16
_CHUNK = 64
_NBUF = 4


def _triplet_loss_body(t_per_w, n_chunks, d,
                       emb_a, emb_p, emb_n, aidx, pidx, nidx, out,
                       *refs):
    bufs = refs[3:3 + 3 * _NBUF]
    aidx_v, pidx_v, nidx_v = refs[0:3]
    vacc_v = refs[3 + 3 * _NBUF]
    sem_i = refs[4 + 3 * _NBUF]
    sems = refs[5 + 3 * _NBUF:]
    bufsets = tuple((bufs[3 * i], bufs[3 * i + 1], bufs[3 * i + 2], sems[i])
                    for i in range(_NBUF))

    wid = lax.axis_index("s") * _NC + lax.axis_index("c")
    base = wid * t_per_w

    ci0 = pltpu.async_copy(aidx.at[pl.ds(base, t_per_w)], aidx_v, sem_i)
    ci1 = pltpu.async_copy(pidx.at[pl.ds(base, t_per_w)], pidx_v, sem_i)
    ci2 = pltpu.async_copy(nidx.at[pl.ds(base, t_per_w)], nidx_v, sem_i)
    ci0.wait()
    ci1.wait()
    ci2.wait()

    def copies(k, bs):
        ab, pb, nb, sem = bs
        off = k * _CHUNK
        return (
            pltpu.make_async_copy(emb_a.at[aidx_v.at[pl.ds(off, _CHUNK)]], ab, sem),
            pltpu.make_async_copy(emb_p.at[pidx_v.at[pl.ds(off, _CHUNK)]], pb, sem),
            pltpu.make_async_copy(emb_n.at[nidx_v.at[pl.ds(off, _CHUNK)]], nb, sem),
        )

    def issue(k, bs):
        for c in copies(k, bs):
            c.start()

    def drain(k, bs):
        for c in copies(k, bs):
            c.wait()

    lane = lax.iota(jnp.int32, _L)
    is_last = lane == _L - 1

    def compute(bs, vacc):
        ab, pb, nb, _ = bs

        def t_body(t, vacc):
            acc = jnp.zeros((_L,), jnp.float32)
            for jg in range(d // _L):
                sl = pl.ds(jg * _L, _L)
                a = ab[t, sl]
                p = pb[t, sl]
                n = nb[t, sl]
                dp = a - p
                dn = a - n
                acc = acc + (dp * dp - dn * dn)
            s = plsc.cumsum(acc)
            contrib = jnp.maximum(s + _MARGIN, 0.0)
            return vacc + jnp.where(is_last, contrib, 0.0)

        return plsc.parallel_loop(0, _CHUNK, unroll=4, carry=vacc)(t_body)

    for b in range(_NBUF - 2):
        issue(b, bufsets[b])

    def round_body(j, vacc):
        k0 = _NBUF * j
        for b in range(_NBUF):
            k = k0 + b
            kn = k + _NBUF - 2
            bn = (b + _NBUF - 2) % _NBUF
            @pl.when(kn < n_chunks)
            def _():
                issue(kn, bufsets[bn])
            drain(k, bufsets[b])
            vacc = compute(bufsets[b], vacc)
        return vacc

    vacc = lax.fori_loop(0, n_chunks // _NBUF, round_body,
                         jnp.zeros((_L,), jnp.float32))
    vacc_v[...] = vacc
    pltpu.sync_copy(vacc_v, out.at[wid])


def kernel(embeddings, target, triplets):
    del target
    t = triplets.shape[0]
    d = embeddings.shape[2]
    t_per_w = t // _NW
    n_chunks = t_per_w // _CHUNK

    mesh = plsc.VectorSubcoreMesh(core_axis_name="c", subcore_axis_name="s",
                                  num_cores=_NC, num_subcores=_NS)
    body = functools.partial(_triplet_loss_body, t_per_w, n_chunks, d)
    run = pl.kernel(
        body,
        out_type=jax.ShapeDtypeStruct((_NW, _L), jnp.float32),
        mesh=mesh,
        compiler_params=pltpu.CompilerParams(needs_layout_passes=False),
        scratch_types=(
            [pltpu.VMEM((t_per_w,), jnp.int32)] * 3
            + [pltpu.VMEM((_CHUNK, d), jnp.float32)] * (3 * _NBUF)
            + [pltpu.VMEM((_L,), jnp.float32)]
            + [pltpu.SemaphoreType.DMA] * (1 + _NBUF)
        ),
    )
    partials = run(embeddings[0], embeddings[1], embeddings[2],
                   triplets[:, 0], triplets[:, 1], triplets[:, 2])
    loss = jnp.sum(partials) / jnp.float32(t)
    return (loss, t)
